# Initial kernel scaffold; baseline (speedup 1.0000x reference)
import jax
import jax.numpy as jnp
from jax.experimental import pallas as pl


def kernel(x):
    raise NotImplementedError("write your pallas kernel here")


# speedup vs baseline: 1.9691x; 1.9691x over previous
import jax
import jax.numpy as jnp
from jax.experimental import pallas as pl

_NUM_REGIONS = 8
_CH_PER_REGION = 16
_BM = 32


def _split_body(x_ref, *out_refs):
    for k in range(_NUM_REGIONS):
        out_refs[k][...] = x_ref[:, k * _CH_PER_REGION:(k + 1) * _CH_PER_REGION, :]


def kernel(x):
    b, c, d = x.shape
    grid = (b // _BM,)
    out_shape = tuple(
        jax.ShapeDtypeStruct((b, _CH_PER_REGION, d), x.dtype)
        for _ in range(_NUM_REGIONS)
    )
    return pl.pallas_call(
        _split_body,
        grid=grid,
        in_specs=[pl.BlockSpec((_BM, c, d), lambda i: (i, 0, 0))],
        out_specs=tuple(
            pl.BlockSpec((_BM, _CH_PER_REGION, d), lambda i: (i, 0, 0))
            for _ in range(_NUM_REGIONS)
        ),
        out_shape=out_shape,
    )(x)


# speedup vs baseline: 1.9726x; 1.0017x over previous
import jax
import jax.numpy as jnp
from jax.experimental import pallas as pl

_NUM_REGIONS = 8
_CH_PER_REGION = 16
_BM = 64


def _split_body(x_ref, *out_refs):
    for k in range(_NUM_REGIONS):
        out_refs[k][...] = x_ref[:, k * _CH_PER_REGION:(k + 1) * _CH_PER_REGION, :]


def kernel(x):
    b, c, d = x.shape
    grid = (b // _BM,)
    out_shape = tuple(
        jax.ShapeDtypeStruct((b, _CH_PER_REGION, d), x.dtype)
        for _ in range(_NUM_REGIONS)
    )
    return pl.pallas_call(
        _split_body,
        grid=grid,
        in_specs=[pl.BlockSpec((_BM, c, d), lambda i: (i, 0, 0))],
        out_specs=tuple(
            pl.BlockSpec((_BM, _CH_PER_REGION, d), lambda i: (i, 0, 0))
            for _ in range(_NUM_REGIONS)
        ),
        out_shape=out_shape,
    )(x)
